# Initial kernel scaffold; baseline (speedup 1.0000x reference)
import jax
import jax.numpy as jnp
from jax import lax
from jax.experimental import pallas as pl
from jax.experimental.pallas import tpu as pltpu
from jax.experimental.pallas import tpu_sc as plsc

DIM = 64
BATCH = 4096
RATERS = 32
NUM_CORES = 2
NUM_SUBCORES = 16
NUM_WORKERS = NUM_CORES * NUM_SUBCORES
BPW = BATCH // NUM_WORKERS
LANES = 16


def _body(v_hbm, iu_hbm, tab_hbm, out_hbm, v_v, raters_v, raters_t, acc_v,
          sem_in, sem_acc):
    wid = lax.axis_index("s") * NUM_CORES + lax.axis_index("c")
    base = wid * BPW

    pltpu.sync_copy(v_hbm.at[pl.ds(base, BPW)], v_v)

    pltpu.async_copy(iu_hbm.at[v_v], raters_v, sem_in).wait()

    lane = lax.iota(jnp.int32, LANES)

    def transpose_row(j, carry):
        jv = jnp.full((LANES,), j, dtype=jnp.int32)
        for half in range(RATERS // LANES):
            chunk = raters_v[j, pl.ds(half * LANES, LANES)]
            plsc.store_scatter(raters_t, [lane + half * LANES, jv], chunk)
        return carry

    lax.fori_loop(0, BPW, transpose_row, 0, unroll=4)

    pltpu.async_copy(tab_hbm.at[raters_t.at[0]], acc_v, sem_acc).wait()
    copies = [
        pltpu.async_copy(tab_hbm.at[raters_t.at[r]], acc_v, sem_acc, add=True)
        for r in range(1, RATERS)
    ]
    for c in copies:
        c.wait()

    pltpu.sync_copy(acc_v, out_hbm.at[pl.ds(base, BPW)])


@jax.jit
def _ucn_sc(v, item_users, U_table):
    mesh = plsc.VectorSubcoreMesh(core_axis_name="c", subcore_axis_name="s")
    return pl.kernel(
        _body,
        out_type=jax.ShapeDtypeStruct((BATCH, DIM), jnp.float32),
        mesh=mesh,
        scratch_types=[
            pltpu.VMEM((BPW,), jnp.int32),
            pltpu.VMEM((BPW, RATERS), jnp.int32),
            pltpu.VMEM((RATERS, BPW), jnp.int32),
            pltpu.VMEM((BPW, DIM), jnp.float32),
            pltpu.SemaphoreType.DMA,
            pltpu.SemaphoreType.DMA,
        ],
    )(v, item_users, U_table)


def kernel(u, v, item_users, U_table):
    del u
    return _ucn_sc(v, item_users, U_table)


# speedup vs baseline: 9.6705x; 9.6705x over previous
import jax
import jax.numpy as jnp
from jax import lax
from jax.experimental import pallas as pl
from jax.experimental.pallas import tpu as pltpu
from jax.experimental.pallas import tpu_sc as plsc

DIM = 64
BATCH = 4096
RATERS = 32
NUM_CORES = 2
NUM_SUBCORES = 16
NUM_WORKERS = NUM_CORES * NUM_SUBCORES
BPW = BATCH // NUM_WORKERS
LANES = 16


def _body(v_hbm, iu_hbm, tab_hbm, out_hbm, v_v, raters_v, raters_t, acc_v,
          sem_in, sem_acc):
    wid = lax.axis_index("s") * NUM_CORES + lax.axis_index("c")
    base = wid * BPW

    pltpu.sync_copy(v_hbm.at[pl.ds(base, BPW)], v_v)

    pltpu.async_copy(iu_hbm.at[v_v], raters_v, sem_in).wait()

    lane = lax.iota(jnp.int32, LANES)

    def transpose_row(j, carry):
        for half in range(RATERS // LANES):
            chunk = raters_v[j, pl.ds(half * LANES, LANES)]
            flat_idx = (lane + half * LANES) * BPW + j
            plsc.store_scatter(raters_t, [flat_idx], chunk)
        return carry

    lax.fori_loop(0, BPW, transpose_row, 0, unroll=4)

    pltpu.async_copy(
        tab_hbm.at[raters_t.at[pl.ds(0, BPW)]], acc_v, sem_acc).wait()
    copies = [
        pltpu.async_copy(
            tab_hbm.at[raters_t.at[pl.ds(r * BPW, BPW)]], acc_v, sem_acc,
            add=True)
        for r in range(1, RATERS)
    ]
    for c in copies:
        c.wait()

    pltpu.sync_copy(acc_v, out_hbm.at[pl.ds(base, BPW)])


@jax.jit
def _ucn_sc(v, item_users, U_table):
    mesh = plsc.VectorSubcoreMesh(core_axis_name="c", subcore_axis_name="s")
    return pl.kernel(
        _body,
        out_type=jax.ShapeDtypeStruct((BATCH, DIM), jnp.float32),
        mesh=mesh,
        compiler_params=pltpu.CompilerParams(
            needs_layout_passes=False, use_tc_tiling_on_sc=False),
        scratch_types=[
            pltpu.VMEM((BPW,), jnp.int32),
            pltpu.VMEM((BPW, RATERS), jnp.int32),
            pltpu.VMEM((RATERS * BPW,), jnp.int32),
            pltpu.VMEM((BPW, DIM), jnp.float32),
            pltpu.SemaphoreType.DMA,
            pltpu.SemaphoreType.DMA,
        ],
    )(v, item_users, U_table)


def kernel(u, v, item_users, U_table):
    del u
    return _ucn_sc(v, item_users, U_table)


# speedup vs baseline: 9.9756x; 1.0316x over previous
import jax
import jax.numpy as jnp
from jax import lax
from jax.experimental import pallas as pl
from jax.experimental.pallas import tpu as pltpu
from jax.experimental.pallas import tpu_sc as plsc

DIM = 64
BATCH = 4096
RATERS = 32
NUM_CORES = 2
NUM_SUBCORES = 16
NUM_WORKERS = NUM_CORES * NUM_SUBCORES
BPW = BATCH // NUM_WORKERS
LANES = 16


def _body(v_hbm, iu_hbm, tab_hbm, out_hbm, v_v, raters_v, raters_t, acc_v,
          sem_in, sem_acc):
    wid = lax.axis_index("s") * NUM_CORES + lax.axis_index("c")
    base = wid * BPW

    pltpu.sync_copy(v_hbm.at[pl.ds(base, BPW)], v_v)

    iu_copy = pltpu.async_copy(iu_hbm.at[v_v], raters_v, sem_in)

    zeros = jnp.zeros((LANES,), jnp.float32)

    def zero_chunk(j, carry):
        for c in range(DIM // LANES):
            acc_v[j, pl.ds(c * LANES, LANES)] = zeros
        return carry

    lax.fori_loop(0, BPW, zero_chunk, 0, unroll=4)
    iu_copy.wait()

    lane = lax.iota(jnp.int32, LANES)
    half_rows = BPW // 2
    copies = []
    for h in range(2):
        j0 = h * half_rows

        def transpose_row(j, carry):
            for half in range(RATERS // LANES):
                chunk = raters_v[j, pl.ds(half * LANES, LANES)]
                flat_idx = (lane + half * LANES) * BPW + j
                plsc.store_scatter(raters_t, [flat_idx], chunk)
            return carry

        lax.fori_loop(j0, j0 + half_rows, transpose_row, 0, unroll=4)
        for r in range(RATERS):
            copies.append(pltpu.async_copy(
                tab_hbm.at[raters_t.at[pl.ds(r * BPW + j0, half_rows)]],
                acc_v.at[pl.ds(j0, half_rows)],
                sem_acc, add=True))

    for c in copies:
        c.wait()

    pltpu.sync_copy(acc_v, out_hbm.at[pl.ds(base, BPW)])


@jax.jit
def _ucn_sc(v, item_users, U_table):
    mesh = plsc.VectorSubcoreMesh(core_axis_name="c", subcore_axis_name="s")
    return pl.kernel(
        _body,
        out_type=jax.ShapeDtypeStruct((BATCH, DIM), jnp.float32),
        mesh=mesh,
        compiler_params=pltpu.CompilerParams(
            needs_layout_passes=False, use_tc_tiling_on_sc=False),
        scratch_types=[
            pltpu.VMEM((BPW,), jnp.int32),
            pltpu.VMEM((BPW, RATERS), jnp.int32),
            pltpu.VMEM((RATERS * BPW,), jnp.int32),
            pltpu.VMEM((BPW, DIM), jnp.float32),
            pltpu.SemaphoreType.DMA,
            pltpu.SemaphoreType.DMA,
        ],
    )(v, item_users, U_table)


def kernel(u, v, item_users, U_table):
    del u
    return _ucn_sc(v, item_users, U_table)
